# baseline (speedup 1.0000x reference)
import functools

import jax
import jax.numpy as jnp
from jax import lax
from jax.experimental import pallas as pl
from jax.experimental.pallas import tpu as pltpu
from jax.experimental.pallas import tpu_sc as plsc

N = 10000
E = 320000
D_IN = 128
D_HID = 16
D_OUT = 7

NC = 2
NS = 16
NW = NC * NS

NP = 10240
NPT = NP // NS
CH = 128
ROWS = E // CH
RPT = ROWS // NW
REM = ROWS - NW * RPT
NBUF = 6
ZR = 128


def _make_spmm(D):
    mesh = plsc.VectorSubcoreMesh(core_axis_name="c", subcore_axis_name="s")

    @functools.partial(
        pl.kernel,
        out_type=jax.ShapeDtypeStruct((NC, NP, D), jnp.float32),
        mesh=mesh,
        compiler_params=pltpu.CompilerParams(use_tc_tiling_on_sc=False),
        scratch_types=[
            pltpu.VMEM((RPT + 1, CH), jnp.int32),
            pltpu.VMEM((RPT + 1, CH), jnp.int32),
            [pltpu.VMEM((CH, D), jnp.float32) for _ in range(NBUF)],
            pltpu.VMEM((ZR, D), jnp.float32),
            pltpu.VMEM_SHARED((NP, D), jnp.float32),
            pltpu.VMEM_SHARED((NP, D), jnp.float32),
            [pltpu.SemaphoreType.DMA for _ in range(NBUF)],
            [pltpu.SemaphoreType.DMA for _ in range(NBUF)],
            [pltpu.SemaphoreType.DMA for _ in range(4)],
        ],
    )
    def spmm(src_hbm, dst_hbm, x_hbm, out_hbm,
             src_v, dst_v, bufs, zbuf, acc, xsp, gsems, ssems, stage_sems):
        c = lax.axis_index("c")
        s = lax.axis_index("s")
        wid = s * NC + c
        extra = wid < REM

        base = wid * RPT + jnp.minimum(wid, REM)
        ds_ = pltpu.async_copy(src_hbm.at[pl.ds(base, RPT)],
                               src_v.at[pl.ds(0, RPT)], stage_sems[1])
        dd = pltpu.async_copy(dst_hbm.at[pl.ds(base, RPT)],
                              dst_v.at[pl.ds(0, RPT)], stage_sems[2])
        dx = pltpu.async_copy(x_hbm.at[pl.ds(s * NPT, NPT)],
                              xsp.at[pl.ds(s * NPT, NPT)], stage_sems[3])

        @pl.when(extra)
        def _():
            pltpu.sync_copy(src_hbm.at[pl.ds(base + RPT, 1)],
                            src_v.at[pl.ds(RPT, 1)])
            pltpu.sync_copy(dst_hbm.at[pl.ds(base + RPT, 1)],
                            dst_v.at[pl.ds(RPT, 1)])

        def zfill(i, carry):
            zbuf[i, :] = jnp.zeros((D,), jnp.float32)
            return carry

        lax.fori_loop(0, ZR, zfill, 0)
        for k in range(NPT // ZR):
            pltpu.async_copy(zbuf, acc.at[pl.ds(s * NPT + k * ZR, ZR)],
                             stage_sems[0])

        def gath(j, b):
            return pltpu.async_copy(xsp.at[src_v.at[j]], bufs[b], gsems[b])

        def gwait(j, b):
            pltpu.make_async_copy(xsp.at[src_v.at[j]], bufs[b],
                                  gsems[b]).wait()

        def scat(j, b):
            return pltpu.async_copy(bufs[b], acc.at[dst_v.at[j]], ssems[b],
                                    add=True)

        def swait(j, b):
            pltpu.make_async_copy(bufs[b], acc.at[dst_v.at[j]],
                                  ssems[b]).wait()

        ds_.wait()
        dd.wait()
        dx.wait()
        for k in range(NPT // ZR):
            pltpu.make_async_copy(
                zbuf, acc.at[pl.ds(s * NPT + k * ZR, ZR)],
                stage_sems[0]).wait()
        plsc.subcore_barrier()
        for b in range(NBUF):
            gath(b, b)

        niter = RPT // NBUF - 1

        def body(i, carry):
            j0 = NBUF * i
            for b in range(NBUF):
                gwait(j0 + b, b)
                scat(j0 + b, b)
            for b in range(NBUF):
                swait(j0 + b, b)
                gath(j0 + b + NBUF, b)
            return carry

        lax.fori_loop(0, niter, body, 0)

        j0 = NBUF * niter
        for b in range(NBUF):
            gwait(j0 + b, b)
            scat(j0 + b, b)
        for b in range(NBUF):
            swait(j0 + b, b)

        @pl.when(extra)
        def _():
            gath(RPT, 0)
            gwait(RPT, 0)
            scat(RPT, 0)
            swait(RPT, 0)

        plsc.subcore_barrier()
        pltpu.sync_copy(acc.at[pl.ds(s * NPT, NPT)],
                        out_hbm.at[c, pl.ds(s * NPT, NPT)])

    return spmm


_spmm16 = _make_spmm(D_HID)


def _mm1_body(x_ref, w_ref, o_ref):
    o_ref[:N, :] = jnp.dot(x_ref[...], w_ref[...],
                           preferred_element_type=jnp.float32)
    o_ref[N:, :] = jnp.zeros((NP - N, D_HID), jnp.float32)


_mm1 = pl.pallas_call(
    _mm1_body,
    out_shape=jax.ShapeDtypeStruct((NP, D_HID), jnp.float32),
)


def _mid_body(p_ref, b1_ref, w2_ref, o_ref):
    h = jnp.maximum(p_ref[0] + p_ref[1] + b1_ref[...], 0.0)
    o_ref[:, :D_OUT] = jnp.dot(h, w2_ref[...],
                               preferred_element_type=jnp.float32)
    o_ref[:, D_OUT:] = jnp.zeros((NP, D_HID - D_OUT), jnp.float32)


_mid = pl.pallas_call(
    _mid_body,
    out_shape=jax.ShapeDtypeStruct((NP, D_HID), jnp.float32),
)


def _fin_body(p_ref, b2_ref, o_ref):
    o_ref[...] = (p_ref[0, :N, :D_OUT] + p_ref[1, :N, :D_OUT]
                  + b2_ref[...])


_fin = pl.pallas_call(
    _fin_body,
    out_shape=jax.ShapeDtypeStruct((N, D_OUT), jnp.float32),
)


def kernel(adjacency, feature, W1, b1, W2, b2):
    src2 = adjacency[0].astype(jnp.int32).reshape(ROWS, CH)
    dst2 = adjacency[1].astype(jnp.int32).reshape(ROWS, CH)

    support1 = _mm1(feature, W1)
    part1 = _spmm16(src2, dst2, support1)
    support2 = _mid(part1, b1.reshape(1, D_HID), W2)
    part2 = _spmm16(src2, dst2, support2)
    logits = _fin(part2, b2.reshape(1, D_OUT))
    return logits


# speedup vs baseline: 1.0107x; 1.0107x over previous
import functools

import jax
import jax.numpy as jnp
from jax import lax
from jax.experimental import pallas as pl
from jax.experimental.pallas import tpu as pltpu
from jax.experimental.pallas import tpu_sc as plsc

N = 10000
E = 320000
D_IN = 128
D_HID = 16
D_OUT = 7

NC = 2
NS = 16
NW = NC * NS

NP = 10240
NPT = NP // NS
CH = 128
ROWS = E // CH
RPT = ROWS // NW
REM = ROWS - NW * RPT
NBUF = 6
ZR = 128


def _make_spmm(D):
    mesh = plsc.VectorSubcoreMesh(core_axis_name="c", subcore_axis_name="s")

    @functools.partial(
        pl.kernel,
        out_type=jax.ShapeDtypeStruct((NC, NP, D), jnp.float32),
        mesh=mesh,
        compiler_params=pltpu.CompilerParams(use_tc_tiling_on_sc=False),
        scratch_types=[
            pltpu.VMEM((RPT + 1, CH), jnp.int32),
            pltpu.VMEM((RPT + 1, CH), jnp.int32),
            [pltpu.VMEM((CH, D), jnp.float32) for _ in range(NBUF)],
            pltpu.VMEM((ZR, D), jnp.float32),
            pltpu.VMEM_SHARED((NP, D), jnp.float32),
            [pltpu.SemaphoreType.DMA for _ in range(NBUF)],
            [pltpu.SemaphoreType.DMA for _ in range(NBUF)],
            [pltpu.SemaphoreType.DMA for _ in range(3)],
        ],
    )
    def spmm(src_hbm, dst_hbm, x_hbm, out_hbm,
             src_v, dst_v, bufs, zbuf, acc, gsems, ssems, stage_sems):
        c = lax.axis_index("c")
        s = lax.axis_index("s")
        wid = s * NC + c
        extra = wid < REM

        base = wid * RPT + jnp.minimum(wid, REM)
        ds_ = pltpu.async_copy(src_hbm.at[pl.ds(base, RPT)],
                               src_v.at[pl.ds(0, RPT)], stage_sems[1])
        dd = pltpu.async_copy(dst_hbm.at[pl.ds(base, RPT)],
                              dst_v.at[pl.ds(0, RPT)], stage_sems[2])

        @pl.when(extra)
        def _():
            pltpu.sync_copy(src_hbm.at[pl.ds(base + RPT, 1)],
                            src_v.at[pl.ds(RPT, 1)])
            pltpu.sync_copy(dst_hbm.at[pl.ds(base + RPT, 1)],
                            dst_v.at[pl.ds(RPT, 1)])

        def zfill(i, carry):
            zbuf[i, :] = jnp.zeros((D,), jnp.float32)
            return carry

        lax.fori_loop(0, ZR, zfill, 0)
        for k in range(NPT // ZR):
            pltpu.async_copy(zbuf, acc.at[pl.ds(s * NPT + k * ZR, ZR)],
                             stage_sems[0])

        def gath(j, b):
            return pltpu.async_copy(x_hbm.at[src_v.at[j]], bufs[b], gsems[b])

        def gwait(j, b):
            pltpu.make_async_copy(x_hbm.at[src_v.at[j]], bufs[b],
                                  gsems[b]).wait()

        def scat(j, b):
            return pltpu.async_copy(bufs[b], acc.at[dst_v.at[j]], ssems[b],
                                    add=True)

        def swait(j, b):
            pltpu.make_async_copy(bufs[b], acc.at[dst_v.at[j]],
                                  ssems[b]).wait()

        ds_.wait()
        for b in range(NBUF):
            gath(b, b)
        dd.wait()
        for k in range(NPT // ZR):
            pltpu.make_async_copy(
                zbuf, acc.at[pl.ds(s * NPT + k * ZR, ZR)],
                stage_sems[0]).wait()
        plsc.subcore_barrier()

        niter = RPT // NBUF - 1

        def body(i, carry):
            j0 = NBUF * i
            for b in range(NBUF):
                gwait(j0 + b, b)
                scat(j0 + b, b)
            for b in range(NBUF):
                swait(j0 + b, b)
                gath(j0 + b + NBUF, b)
            return carry

        lax.fori_loop(0, niter, body, 0)

        j0 = NBUF * niter
        for b in range(NBUF):
            gwait(j0 + b, b)
            scat(j0 + b, b)
        for b in range(NBUF):
            swait(j0 + b, b)

        @pl.when(extra)
        def _():
            gath(RPT, 0)
            gwait(RPT, 0)
            scat(RPT, 0)
            swait(RPT, 0)

        plsc.subcore_barrier()
        pltpu.sync_copy(acc.at[pl.ds(s * NPT, NPT)],
                        out_hbm.at[c, pl.ds(s * NPT, NPT)])

    return spmm


_spmm16 = _make_spmm(D_HID)


def _mm1_body(x_ref, w_ref, o_ref):
    o_ref[...] = jnp.dot(x_ref[...], w_ref[...],
                         preferred_element_type=jnp.float32)


_mm1 = pl.pallas_call(
    _mm1_body,
    out_shape=jax.ShapeDtypeStruct((N, D_HID), jnp.float32),
)


def _mid_body(p_ref, b1_ref, w2_ref, o_ref):
    h = jnp.maximum(p_ref[0] + p_ref[1] + b1_ref[...], 0.0)
    o_ref[:, :D_OUT] = jnp.dot(h, w2_ref[...],
                               preferred_element_type=jnp.float32)
    o_ref[:, D_OUT:] = jnp.zeros((NP, D_HID - D_OUT), jnp.float32)


_mid = pl.pallas_call(
    _mid_body,
    out_shape=jax.ShapeDtypeStruct((NP, D_HID), jnp.float32),
)


def _fin_body(p_ref, b2_ref, o_ref):
    o_ref[...] = (p_ref[0, :N, :D_OUT] + p_ref[1, :N, :D_OUT]
                  + b2_ref[...])


_fin = pl.pallas_call(
    _fin_body,
    out_shape=jax.ShapeDtypeStruct((N, D_OUT), jnp.float32),
)


def kernel(adjacency, feature, W1, b1, W2, b2):
    src2 = adjacency[0].astype(jnp.int32).reshape(ROWS, CH)
    dst2 = adjacency[1].astype(jnp.int32).reshape(ROWS, CH)

    support1 = _mm1(feature, W1)
    part1 = _spmm16(src2, dst2, support1)
    support2 = _mid(part1, b1.reshape(1, D_HID), W2)
    part2 = _spmm16(src2, dst2, support2)
    logits = _fin(part2, b2.reshape(1, D_OUT))
    return logits
